# baseline (device time: 17666 ns/iter reference)
import jax
import jax.numpy as jnp
from jax import lax
from jax.experimental import pallas as pl
from jax.experimental.pallas import tpu as pltpu

N_DEV = 8
H = 4


def kernel(A, B):
    m, k = A.shape
    k2, n = B.shape
    mc = m // N_DEV
    hc = mc // H

    def body(a_ref, b_ref, out_ref, part_ref, *rest):
        p1_q = rest[0:H]
        g_q = rest[H:2 * H]
        send_p1 = rest[2 * H:3 * H]
        recv_p1 = rest[3 * H:4 * H]
        send_p2 = rest[4 * H:5 * H]
        recv_p2 = rest[5 * H:6 * H]
        my = lax.axis_index("i")

        barrier_sem = pltpu.get_barrier_semaphore()
        for d in range(1, N_DEV):
            pl.semaphore_signal(
                barrier_sem, inc=1,
                device_id=((my + d) % N_DEV,),
                device_id_type=pl.DeviceIdType.MESH,
            )

        part_ref[:, :, :, :] = jnp.dot(
            a_ref[:, :].astype(jnp.bfloat16),
            b_ref[:, :].astype(jnp.bfloat16),
            preferred_element_type=jnp.float32,
        ).astype(jnp.bfloat16).reshape(N_DEV, H, hc, n)

        pl.semaphore_wait(barrier_sem, N_DEV - 1)

        p1_sends = []
        for h in range(H):
            for d in range(1, N_DEV):
                tgt = (my + d) % N_DEV
                rdma = pltpu.make_async_remote_copy(
                    src_ref=part_ref.at[tgt, h],
                    dst_ref=p1_q[h].at[my],
                    send_sem=send_p1[h].at[d - 1],
                    recv_sem=recv_p1[h].at[my],
                    device_id=(tgt,),
                    device_id_type=pl.DeviceIdType.MESH,
                )
                rdma.start()
                p1_sends.append(rdma)

        p2_sends = []
        for h in range(H):
            z = part_ref[pl.ds(my, 1), h].astype(jnp.float32)
            for d in range(1, N_DEV):
                src = (my + d) % N_DEV
                recv = pltpu.make_async_remote_copy(
                    src_ref=p1_q[h].at[src],
                    dst_ref=p1_q[h].at[src],
                    send_sem=send_p1[h].at[d - 1],
                    recv_sem=recv_p1[h].at[src],
                    device_id=(src,),
                    device_id_type=pl.DeviceIdType.MESH,
                )
                recv.wait_recv()
                z += p1_q[h][pl.ds(src, 1), :, :].astype(jnp.float32)
            z = z[0]
            silu = z / (1.0 + jnp.exp(-z))
            g_q[h][pl.ds(my, 1), :, :] = silu.astype(jnp.bfloat16)[None]
            out_ref[pl.ds(my * mc + h * hc, hc), :] = silu
            for d in range(1, N_DEV):
                tgt = (my + d) % N_DEV
                rdma = pltpu.make_async_remote_copy(
                    src_ref=g_q[h].at[my],
                    dst_ref=g_q[h].at[my],
                    send_sem=send_p2[h].at[d - 1],
                    recv_sem=recv_p2[h].at[my],
                    device_id=(tgt,),
                    device_id_type=pl.DeviceIdType.MESH,
                )
                rdma.start()
                p2_sends.append(rdma)

        for rdma in p1_sends:
            rdma.wait_send()

        for h in range(H):
            for d in range(1, N_DEV):
                src = (my + d) % N_DEV
                recv = pltpu.make_async_remote_copy(
                    src_ref=g_q[h].at[src],
                    dst_ref=g_q[h].at[src],
                    send_sem=send_p2[h].at[d - 1],
                    recv_sem=recv_p2[h].at[src],
                    device_id=(src,),
                    device_id_type=pl.DeviceIdType.MESH,
                )
                recv.wait_recv()
                out_ref[pl.ds(src * mc + h * hc, hc), :] = (
                    g_q[h][pl.ds(src, 1), :, :].astype(jnp.float32)[0]
                )

        for rdma in p2_sends:
            rdma.wait_send()

    qbuf = pltpu.VMEM((N_DEV, m // N_DEV // H, n), jnp.bfloat16)
    return pl.pallas_call(
        body,
        out_shape=jax.ShapeDtypeStruct((m, n), jnp.float32),
        in_specs=[
            pl.BlockSpec(memory_space=pltpu.VMEM),
            pl.BlockSpec(memory_space=pltpu.VMEM),
        ],
        out_specs=pl.BlockSpec(memory_space=pltpu.VMEM),
        scratch_shapes=(
            [pltpu.VMEM((N_DEV, H, m // N_DEV // H, n), jnp.bfloat16)]
            + [qbuf] * H
            + [qbuf] * H
            + [pltpu.SemaphoreType.DMA((N_DEV - 1,))] * H
            + [pltpu.SemaphoreType.DMA((N_DEV,))] * H
            + [pltpu.SemaphoreType.DMA((N_DEV - 1,))] * H
            + [pltpu.SemaphoreType.DMA((N_DEV,))] * H
        ),
        compiler_params=pltpu.CompilerParams(collective_id=0),
    )(A, B)
